# baseline (speedup 1.0000x reference)
import jax
import jax.numpy as jnp
from jax.experimental import pallas as pl
from jax.experimental.pallas import tpu as pltpu

_SCORE_THR = 0.05
_IOU_THR = 0.5
_N = 5000
_NPAD = 5120
_BT = 256
_NB = _NPAD // _BT
_CAP = 96


def _nms_kernel(data_c_ref, data_r_ref, out_ref,
                keep_ref, delta_ref, acc_ref, flag_ref, cache_ref,
                slot_base_ref, tmin_ref, tmax_ref, band_lo_ref, band_hi_ref):
    n = _NPAD

    scores_row = data_r_ref[4:5, :]
    valid = (scores_row >= _SCORE_THR).astype(jnp.float32)
    keep_ref[0:1, :] = valid
    delta_ref[0:1, :] = valid
    acc_ref[0:1, :] = jnp.zeros((1, n), jnp.float32)

    def tile_minmax(jb, c):
        cls_slice = data_r_ref[5:6, pl.ds(jb * _BT, _BT)]
        tmin_ref[jb] = jnp.min(cls_slice)
        tmax_ref[jb] = jnp.max(cls_slice)
        return c

    jax.lax.fori_loop(0, _NB, tile_minmax, 0)

    def init_bands(jb, base):
        lo_j = tmin_ref[jb]
        hi_j = tmax_ref[jb]

        def count(k, lohi):
            lo, hi = lohi
            lo = lo + jnp.where(tmax_ref[k] < lo_j, 1, 0)
            hi = hi - jnp.where(tmin_ref[k] > hi_j, 1, 0)
            return lo, hi

        lo, hi = jax.lax.fori_loop(
            0, _NB, count, (jnp.int32(0), jnp.int32(_NB)))
        band_lo_ref[jb] = jnp.maximum(lo, jb)
        band_hi_ref[jb] = hi
        flag_ref[jb] = 1.0
        slot_base_ref[jb] = base
        cnt = jnp.maximum(hi - jnp.maximum(lo, jb), 0)
        return base + cnt

    jax.lax.fori_loop(0, _NB, init_bands, jnp.int32(0))

    max_coord = jnp.max(data_r_ref[0:4, :])
    off_scale = max_coord + 1.0

    jpos = jax.lax.broadcasted_iota(jnp.int32, (_BT, 1), 0)
    ipos = jax.lax.broadcasted_iota(jnp.int32, (1, _BT), 1)

    def sweep(state):
        _, t = state

        def jb_body(jb, carry):
            @pl.when(flag_ref[jb] != 0.0)
            def _():
                j0 = jb * _BT
                ib_start = band_lo_ref[jb]
                ib_end = band_hi_ref[jb]
                base = slot_base_ref[jb]
                all_cached = (t > 0) & (base + (ib_end - ib_start) <= _CAP)

                dj = delta_ref[0:1, pl.ds(j0, _BT)]
                dj8 = jnp.broadcast_to(dj, (8, _BT))

                def cached_path():
                    def ib_cached(ib, c):
                        i0 = ib * _BT
                        slot = base + (ib - ib_start)
                        sf = cache_ref[pl.ds(slot, 1), :, :][0]
                        contrib = jax.lax.dot(
                            dj8, sf, preferred_element_type=jnp.float32)
                        acc_ref[0:1, pl.ds(i0, _BT)] += contrib[0:1, :]
                        return c

                    jax.lax.fori_loop(ib_start, ib_end, ib_cached, 0)

                def full_path():
                    cj_all = data_c_ref[pl.ds(j0, _BT), :]
                    offj = cj_all[:, 5:6] * off_scale
                    shape = (_BT, _BT)
                    xj1 = jnp.broadcast_to(cj_all[:, 0:1] + offj, shape)
                    yj1 = jnp.broadcast_to(cj_all[:, 1:2] + offj, shape)
                    xj2 = jnp.broadcast_to(cj_all[:, 2:3] + offj, shape)
                    yj2 = jnp.broadcast_to(cj_all[:, 3:4] + offj, shape)
                    aj = (xj2 - xj1 + 1.0) * (yj2 - yj1 + 1.0)

                    def ib_body(ib, c):
                        i0 = ib * _BT
                        slot = base + (ib - ib_start)

                        def compute_sf():
                            offi = data_r_ref[5:6, pl.ds(i0, _BT)] * off_scale
                            xi1 = data_r_ref[0:1, pl.ds(i0, _BT)] + offi
                            yi1 = data_r_ref[1:2, pl.ds(i0, _BT)] + offi
                            xi2 = data_r_ref[2:3, pl.ds(i0, _BT)] + offi
                            yi2 = data_r_ref[3:4, pl.ds(i0, _BT)] + offi
                            ai = (xi2 - xi1 + 1.0) * (yi2 - yi1 + 1.0)

                            xmin = jnp.maximum(xj1, xi1)
                            ymin = jnp.maximum(yj1, yi1)
                            xmax = jnp.minimum(xj2, xi2)
                            ymax = jnp.minimum(yj2, yi2)
                            inter = (jnp.maximum(xmax - xmin, 0.0)
                                     * jnp.maximum(ymax - ymin, 0.0))
                            iou = inter / (aj + ai - inter)
                            off_diag = ib != jb
                            prec = off_diag | (jpos < ipos)
                            return ((iou > _IOU_THR) & prec).astype(jnp.float32)

                        def first_sweep():
                            sf = compute_sf()

                            @pl.when(slot < _CAP)
                            def _():
                                cache_ref[pl.ds(slot, 1), :, :] = sf[None]

                            return sf

                        def later_sweep():
                            return jax.lax.cond(
                                slot < _CAP,
                                lambda: cache_ref[pl.ds(slot, 1), :, :][0],
                                compute_sf)

                        sf = jax.lax.cond(t == 0, first_sweep, later_sweep)

                        contrib = jax.lax.dot(
                            dj8, sf, preferred_element_type=jnp.float32)
                        acc_ref[0:1, pl.ds(i0, _BT)] += contrib[0:1, :]
                        return c

                    jax.lax.fori_loop(ib_start, ib_end, ib_body, 0)

                jax.lax.cond(all_cached, cached_path, full_path)

            return carry

        jax.lax.fori_loop(0, _NB, jb_body, 0)

        old = keep_ref[0:1, :]
        new = valid * (acc_ref[0:1, :] < 0.5).astype(jnp.float32)
        delta = new - old
        keep_ref[0:1, :] = new
        delta_ref[0:1, :] = delta

        def set_flags(jb, c):
            flag_ref[jb] = jnp.max(jnp.abs(delta_ref[0:1, pl.ds(jb * _BT, _BT)]))
            return c

        jax.lax.fori_loop(0, _NB, set_flags, 0)
        changed = jnp.max(jnp.abs(delta)) > 0.0
        return changed, t + 1

    jax.lax.while_loop(lambda s: s[0] & (s[1] < n + 2), sweep,
                       (True, jnp.int32(0)))

    out_ref[0:1, :] = keep_ref[0:1, :]


def _nms_call(data_c, data_r, interpret=False):
    return pl.pallas_call(
        _nms_kernel,
        out_shape=jax.ShapeDtypeStruct((1, _NPAD), jnp.float32),
        in_specs=[
            pl.BlockSpec(),
            pl.BlockSpec(),
        ],
        scratch_shapes=[
            pltpu.VMEM((8, _NPAD), jnp.float32),
            pltpu.VMEM((8, _NPAD), jnp.float32),
            pltpu.VMEM((8, _NPAD), jnp.float32),
            pltpu.SMEM((_NB,), jnp.float32),
            pltpu.VMEM((_CAP, _BT, _BT), jnp.float32),
            pltpu.SMEM((_NB,), jnp.int32),
            pltpu.SMEM((_NB,), jnp.float32),
            pltpu.SMEM((_NB,), jnp.float32),
            pltpu.SMEM((_NB,), jnp.int32),
            pltpu.SMEM((_NB,), jnp.int32),
        ],
        interpret=interpret,
    )(data_c, data_r)


def _prep(boxes, scores, class_ids):
    perm = jnp.lexsort((-scores, class_ids))
    data = jnp.concatenate(
        [boxes, scores[:, None], class_ids.astype(jnp.float32)[:, None]],
        axis=1)
    datap = data[perm]

    npad = _NPAD - _N
    pad_row = jnp.array([[0.0, 0.0, 0.0, 0.0, -1.0, 81.0]], jnp.float32)
    data_c = jnp.concatenate(
        [datap, jnp.broadcast_to(pad_row, (npad, 6))], axis=0)
    data_r = data_c.T
    return data_c, data_r, perm


def kernel(boxes, scores, class_ids):
    data_c, data_r, perm = _prep(boxes, scores, class_ids)
    out = _nms_call(data_c, data_r)
    kperm = out[0, :_N]
    k = jnp.zeros((_N,), jnp.float32).at[perm].set(kperm)
    return jnp.concatenate([boxes * k[:, None], (scores * k)[:, None]],
                           axis=1)


# speedup vs baseline: 1.0531x; 1.0531x over previous
import jax
import jax.numpy as jnp
from jax.experimental import pallas as pl
from jax.experimental.pallas import tpu as pltpu

_SCORE_THR = 0.05
_IOU_THR = 0.5
_N = 5000
_NPAD = 5120
_BT = 256
_NB = _NPAD // _BT
_CAP = 96


def _nms_kernel(band_lo_ref, band_hi_ref, data_c_ref, data_r_ref, out_ref,
                keep_ref, delta_ref, acc_ref, flag_ref, cache_ref,
                slot_base_ref):
    n = _NPAD

    scores_row = data_r_ref[4:5, :]
    valid = (scores_row >= _SCORE_THR).astype(jnp.float32)
    keep_ref[0:1, :] = valid
    delta_ref[0:1, :] = valid
    acc_ref[0:1, :] = jnp.zeros((1, n), jnp.float32)

    def init_flags(jb, base):
        flag_ref[jb] = 1.0
        slot_base_ref[jb] = base
        cnt = jnp.maximum(band_hi_ref[jb] - band_lo_ref[jb], 0)
        return base + cnt

    jax.lax.fori_loop(0, _NB, init_flags, jnp.int32(0))

    max_coord = jnp.max(data_r_ref[0:4, :])
    off_scale = max_coord + 1.0

    jpos = jax.lax.broadcasted_iota(jnp.int32, (_BT, 1), 0)
    ipos = jax.lax.broadcasted_iota(jnp.int32, (1, _BT), 1)

    def sweep(state):
        _, t = state

        def jb_body(jb, carry):
            @pl.when(flag_ref[jb] != 0.0)
            def _():
                j0 = jb * _BT
                ib_start = band_lo_ref[jb]
                ib_end = band_hi_ref[jb]
                base = slot_base_ref[jb]
                all_cached = (t > 0) & (base + (ib_end - ib_start) <= _CAP)

                dj = delta_ref[0:1, pl.ds(j0, _BT)]
                dj8 = jnp.broadcast_to(dj, (8, _BT))

                def cached_path():
                    def ib_cached(ib, c):
                        i0 = ib * _BT
                        slot = base + (ib - ib_start)
                        sf = cache_ref[pl.ds(slot, 1), :, :][0]
                        contrib = jax.lax.dot(
                            dj8, sf, preferred_element_type=jnp.float32)
                        acc_ref[0:1, pl.ds(i0, _BT)] += contrib[0:1, :]
                        return c

                    jax.lax.fori_loop(ib_start, ib_end, ib_cached, 0)

                def full_path():
                    cj_all = data_c_ref[pl.ds(j0, _BT), :]
                    offj = cj_all[:, 5:6] * off_scale
                    shape = (_BT, _BT)
                    xj1 = jnp.broadcast_to(cj_all[:, 0:1] + offj, shape)
                    yj1 = jnp.broadcast_to(cj_all[:, 1:2] + offj, shape)
                    xj2 = jnp.broadcast_to(cj_all[:, 2:3] + offj, shape)
                    yj2 = jnp.broadcast_to(cj_all[:, 3:4] + offj, shape)
                    aj = (xj2 - xj1 + 1.0) * (yj2 - yj1 + 1.0)

                    def ib_body(ib, c):
                        i0 = ib * _BT
                        slot = base + (ib - ib_start)

                        def compute_sf():
                            offi = data_r_ref[5:6, pl.ds(i0, _BT)] * off_scale
                            xi1 = data_r_ref[0:1, pl.ds(i0, _BT)] + offi
                            yi1 = data_r_ref[1:2, pl.ds(i0, _BT)] + offi
                            xi2 = data_r_ref[2:3, pl.ds(i0, _BT)] + offi
                            yi2 = data_r_ref[3:4, pl.ds(i0, _BT)] + offi
                            ai = (xi2 - xi1 + 1.0) * (yi2 - yi1 + 1.0)

                            xmin = jnp.maximum(xj1, xi1)
                            ymin = jnp.maximum(yj1, yi1)
                            xmax = jnp.minimum(xj2, xi2)
                            ymax = jnp.minimum(yj2, yi2)
                            inter = (jnp.maximum(xmax - xmin, 0.0)
                                     * jnp.maximum(ymax - ymin, 0.0))
                            iou = inter / (aj + ai - inter)
                            off_diag = ib != jb
                            prec = off_diag | (jpos < ipos)
                            return ((iou > _IOU_THR) & prec).astype(jnp.float32)

                        def first_sweep():
                            sf = compute_sf()

                            @pl.when(slot < _CAP)
                            def _():
                                cache_ref[pl.ds(slot, 1), :, :] = sf[None]

                            return sf

                        def later_sweep():
                            return jax.lax.cond(
                                slot < _CAP,
                                lambda: cache_ref[pl.ds(slot, 1), :, :][0],
                                compute_sf)

                        sf = jax.lax.cond(t == 0, first_sweep, later_sweep)

                        contrib = jax.lax.dot(
                            dj8, sf, preferred_element_type=jnp.float32)
                        acc_ref[0:1, pl.ds(i0, _BT)] += contrib[0:1, :]
                        return c

                    jax.lax.fori_loop(ib_start, ib_end, ib_body, 0)

                jax.lax.cond(all_cached, cached_path, full_path)

            return carry

        jax.lax.fori_loop(0, _NB, jb_body, 0)

        old = keep_ref[0:1, :]
        new = valid * (acc_ref[0:1, :] < 0.5).astype(jnp.float32)
        delta = new - old
        keep_ref[0:1, :] = new
        delta_ref[0:1, :] = delta

        def set_flags(jb, c):
            flag_ref[jb] = jnp.max(jnp.abs(delta_ref[0:1, pl.ds(jb * _BT, _BT)]))
            return c

        jax.lax.fori_loop(0, _NB, set_flags, 0)
        changed = jnp.max(jnp.abs(delta)) > 0.0
        return changed, t + 1

    jax.lax.while_loop(lambda s: s[0] & (s[1] < n + 2), sweep,
                       (True, jnp.int32(0)))

    out_ref[0:1, :] = keep_ref[0:1, :]


def _nms_call(band_lo, band_hi, data_c, data_r, interpret=False):
    return pl.pallas_call(
        _nms_kernel,
        out_shape=jax.ShapeDtypeStruct((1, _NPAD), jnp.float32),
        in_specs=[
            pl.BlockSpec(memory_space=pltpu.SMEM),
            pl.BlockSpec(memory_space=pltpu.SMEM),
            pl.BlockSpec(),
            pl.BlockSpec(),
        ],
        scratch_shapes=[
            pltpu.VMEM((8, _NPAD), jnp.float32),
            pltpu.VMEM((8, _NPAD), jnp.float32),
            pltpu.VMEM((8, _NPAD), jnp.float32),
            pltpu.SMEM((_NB,), jnp.float32),
            pltpu.VMEM((_CAP, _BT, _BT), jnp.float32),
            pltpu.SMEM((_NB,), jnp.int32),
        ],
        interpret=interpret,
    )(band_lo, band_hi, data_c, data_r)


def _prep(boxes, scores, class_ids):
    perm = jnp.lexsort((-scores, class_ids))
    data = jnp.concatenate(
        [boxes, scores[:, None], class_ids.astype(jnp.float32)[:, None]],
        axis=1)
    datap = data[perm]

    npad = _NPAD - _N
    pad_row = jnp.array([[0.0, 0.0, 0.0, 0.0, -1.0, 81.0]], jnp.float32)
    data_c = jnp.concatenate(
        [datap, jnp.broadcast_to(pad_row, (npad, 6))], axis=0)
    data_r = data_c.T

    ci = data_c[:, 5].astype(jnp.int32).reshape(_NB, _BT)
    tmin = ci.min(axis=1)
    tmax = ci.max(axis=1)
    band_lo = jnp.sum(tmax[None, :] < tmin[:, None], axis=1,
                      dtype=jnp.int32)
    band_lo = jnp.maximum(band_lo, jnp.arange(_NB, dtype=jnp.int32))
    band_hi = _NB - jnp.sum(tmin[None, :] > tmax[:, None], axis=1,
                            dtype=jnp.int32)
    return band_lo, band_hi, data_c, data_r, perm


def kernel(boxes, scores, class_ids):
    band_lo, band_hi, data_c, data_r, perm = _prep(boxes, scores, class_ids)
    out = _nms_call(band_lo, band_hi, data_c, data_r)
    kperm = out[0, :_N]
    k = jnp.zeros((_N,), jnp.float32).at[perm].set(kperm)
    return jnp.concatenate([boxes * k[:, None], (scores * k)[:, None]],
                           axis=1)
